# baseline (speedup 1.0000x reference)
import functools

import jax
import jax.numpy as jnp
from jax import lax
from jax.experimental import pallas as pl
from jax.experimental.pallas import tpu as pltpu
from jax.experimental.pallas import tpu_sc as plsc

N = 10000
E = 320000
D = 128
NPAD = 10240
NC, NS = 2, 16
NW = NC * NS
EPW = E // NW
CHUNK = 128
NCH = -(-EPW // CHUNK)
EPWP = NCH * CHUNK
RPS = NPAD // NS

_MESH = plsc.VectorSubcoreMesh(core_axis_name="c", subcore_axis_name="s")


def _zero_f32_buf(buf, nrows, ncols):
    def zr(i, _):
        for j in range(ncols // 16):
            buf[i, pl.ds(j * 16, 16)] = jnp.zeros((16,), jnp.float32)
        return _
    lax.fori_loop(0, nrows, zr, None)


def _prop_body(g_hbm, row_hbm, col_hbm, out_hbm, acc, ri, ci_a, ci_b,
               rows_a, rows_b, gsem_a, gsem_b, csem_a, csem_b):
    cid = lax.axis_index("c")
    sid = lax.axis_index("s")
    w = cid * NS + sid

    _zero_f32_buf(rows_a, CHUNK, D)
    base = sid * RPS
    for k in range(RPS // CHUNK):
        pltpu.sync_copy(rows_a, acc.at[pl.ds(base + k * CHUNK, CHUNK)])

    pltpu.sync_copy(row_hbm.at[w], ri)

    plsc.subcore_barrier()

    def gstart(c, rows, sem):
        cc = jnp.minimum(c, NCH - 1)
        return pltpu.async_copy(g_hbm.at[ri.at[cc]], rows, sem)

    def gwait(c, rows, sem):
        cc = jnp.minimum(c, NCH - 1)
        pltpu.make_async_copy(g_hbm.at[ri.at[cc]], rows, sem).wait()

    def cstart(c, buf, sem):
        cc = jnp.minimum(c, NCH - 1)
        return pltpu.async_copy(col_hbm.at[w, cc], buf, sem)

    def cwait(c, buf, sem):
        cc = jnp.minimum(c, NCH - 1)
        pltpu.make_async_copy(col_hbm.at[w, cc], buf, sem).wait()

    def scat(rows, ci):
        pltpu.sync_copy(rows, acc.at[ci.at[0]], add=True)

    cstart(0, ci_a, csem_a)
    cstart(1, ci_b, csem_b)
    gstart(0, rows_a, gsem_a)
    gstart(1, rows_b, gsem_b)

    def body(cc, carry):
        e = 2 * cc
        gwait(e, rows_a, gsem_a)
        gwait(e + 1, rows_b, gsem_b)
        cwait(e, ci_a, csem_a)
        scat(rows_a, ci_a)
        cwait(e + 1, ci_b, csem_b)
        scat(rows_b, ci_b)
        cstart(e + 2, ci_a, csem_a)
        cstart(e + 3, ci_b, csem_b)
        gstart(e + 2, rows_a, gsem_a)
        gstart(e + 3, rows_b, gsem_b)
        return carry

    lax.fori_loop(0, NCH // 2, body, None)

    gwait(NCH - 1, rows_a, gsem_a)
    cwait(NCH - 1, ci_a, csem_a)
    scat(rows_a, ci_a)
    gwait(NCH, rows_b, gsem_b)
    cwait(NCH, ci_b, csem_b)

    plsc.subcore_barrier()

    pltpu.sync_copy(acc.at[pl.ds(base, RPS)], out_hbm.at[cid, pl.ds(base, RPS)])


@functools.partial(
    pl.kernel,
    out_type=jax.ShapeDtypeStruct((NC, NPAD, D), jnp.float32),
    mesh=_MESH,
    scratch_types=[
        pltpu.VMEM_SHARED((NPAD, D), jnp.float32),
        pltpu.VMEM((NCH, CHUNK), jnp.int32),
        pltpu.VMEM((1, CHUNK), jnp.int32),
        pltpu.VMEM((1, CHUNK), jnp.int32),
        pltpu.VMEM((CHUNK, D), jnp.float32),
        pltpu.VMEM((CHUNK, D), jnp.float32),
        pltpu.SemaphoreType.DMA,
        pltpu.SemaphoreType.DMA,
        pltpu.SemaphoreType.DMA,
        pltpu.SemaphoreType.DMA,
    ],
)
def _sc_propagate(g_hbm, row_hbm, col_hbm, out_hbm, acc, ri, ci_a, ci_b,
                  rows_a, rows_b, gsem_a, gsem_b, csem_a, csem_b):
    _prop_body(g_hbm, row_hbm, col_hbm, out_hbm, acc, ri, ci_a, ci_b,
               rows_a, rows_b, gsem_a, gsem_b, csem_a, csem_b)


def _deg_body(row_hbm, out_hbm, dacc, ri, ones_v, zbuf):
    cid = lax.axis_index("c")
    sid = lax.axis_index("s")
    w = cid * NS + sid

    _zero_f32_buf(zbuf, 128, 16)
    base = sid * RPS
    for k in range(RPS // 128):
        pltpu.sync_copy(zbuf, dacc.at[pl.ds(base + k * 128, 128)])

    def fill(i, _):
        ones_v[i, pl.ds(0, 16)] = jnp.ones((16,), jnp.float32)
        return _
    lax.fori_loop(0, CHUNK, fill, None)

    pltpu.sync_copy(row_hbm.at[w], ri)

    plsc.subcore_barrier()

    def body(c, carry):
        pltpu.sync_copy(ones_v, dacc.at[ri.at[c]], add=True)
        return carry

    lax.fori_loop(0, NCH, body, None)

    plsc.subcore_barrier()

    pltpu.sync_copy(dacc.at[pl.ds(base, RPS)], out_hbm.at[cid, pl.ds(base, RPS)])


@functools.partial(
    pl.kernel,
    out_type=jax.ShapeDtypeStruct((NC, NPAD, 16), jnp.float32),
    mesh=_MESH,
    scratch_types=[
        pltpu.VMEM_SHARED((NPAD, 16), jnp.float32),
        pltpu.VMEM((NCH, CHUNK), jnp.int32),
        pltpu.VMEM((CHUNK, 16), jnp.float32),
        pltpu.VMEM((128, 16), jnp.float32),
    ],
)
def _sc_degree(row_hbm, out_hbm, dacc, ri, ones_v, zbuf):
    _deg_body(row_hbm, out_hbm, dacc, ri, ones_v, zbuf)


BLK = 2048
GRID = NPAD // BLK


def _row_spec():
    return pl.BlockSpec((BLK, D), lambda i: (i, 0))


def _full_spec():
    return pl.BlockSpec((D, D), lambda i: (0, 0))


def _bias_spec():
    return pl.BlockSpec((1, D), lambda i: (0, 0))


def _pair_spec():
    return pl.BlockSpec((NC, BLK, D), lambda i: (0, i, 0))


def _lin_body(x_ref, wt_ref, b_ref, s_ref, o_ref):
    xl = jnp.dot(x_ref[...], wt_ref[...], preferred_element_type=jnp.float32)
    o_ref[...] = s_ref[...] * (xl + b_ref[...])


_tc_lin = pl.pallas_call(
    _lin_body,
    grid=(GRID,),
    in_specs=[_row_spec(), _full_spec(), _bias_spec(), _row_spec()],
    out_specs=_row_spec(),
    out_shape=jax.ShapeDtypeStruct((NPAD, D), jnp.float32),
)


def _mid_body(p_ref, g0_ref, s_ref, h_ref, g1_ref):
    t = p_ref[0] + p_ref[1] + g0_ref[...]
    h = s_ref[...] * t
    h_ref[...] = h
    g1_ref[...] = s_ref[...] * h


_tc_mid = pl.pallas_call(
    _mid_body,
    grid=(GRID,),
    in_specs=[_pair_spec(), _row_spec(), _row_spec()],
    out_specs=[_row_spec(), _row_spec()],
    out_shape=[
        jax.ShapeDtypeStruct((NPAD, D), jnp.float32),
        jax.ShapeDtypeStruct((NPAD, D), jnp.float32),
    ],
)


def _relu_lin_body(hw_ref, p_ref, g1_ref, h1_ref, s_ref, wt_ref, b_ref, o_ref):
    h2 = s_ref[...] * (p_ref[0] + p_ref[1] + g1_ref[...])
    o = hw_ref[0] * h1_ref[...] + hw_ref[1] * h2
    o = jnp.maximum(o, 0.0)
    xl = jnp.dot(o, wt_ref[...], preferred_element_type=jnp.float32)
    o_ref[...] = s_ref[...] * (xl + b_ref[...])


_tc_relu_lin = pl.pallas_call(
    _relu_lin_body,
    grid=(GRID,),
    in_specs=[
        pl.BlockSpec(memory_space=pltpu.SMEM),
        _pair_spec(),
        _row_spec(),
        _row_spec(),
        _row_spec(),
        _full_spec(),
        _bias_spec(),
    ],
    out_specs=_row_spec(),
    out_shape=jax.ShapeDtypeStruct((NPAD, D), jnp.float32),
)


def _final_body(hw_ref, p_ref, g1_ref, h1_ref, s_ref, o_ref):
    h2 = s_ref[...] * (p_ref[0] + p_ref[1] + g1_ref[...])
    o_ref[...] = hw_ref[0] * h1_ref[...] + hw_ref[1] * h2


_tc_final = pl.pallas_call(
    _final_body,
    grid=(GRID,),
    in_specs=[
        pl.BlockSpec(memory_space=pltpu.SMEM),
        _pair_spec(),
        _row_spec(),
        _row_spec(),
        _row_spec(),
    ],
    out_specs=_row_spec(),
    out_shape=jax.ShapeDtypeStruct((NPAD, D), jnp.float32),
)


def kernel(x, edge_index, W1, b1, hw1, W2, b2, hw2):
    pad = jnp.full((NW, EPWP - EPW), N, dtype=jnp.int32)
    row_p = jnp.concatenate([edge_index[0].reshape(NW, EPW), pad], axis=1)
    col_p = jnp.concatenate([edge_index[1].reshape(NW, EPW), pad], axis=1)
    row3 = row_p.reshape(NW, NCH, CHUNK)
    col4 = col_p.reshape(NW, NCH, 1, CHUNK)

    xp = jnp.pad(x, ((0, NPAD - N), (0, 0)))
    b1r = b1.reshape(1, D)
    b2r = b2.reshape(1, D)
    wt1 = W1.T
    wt2 = W2.T

    dparts = _sc_degree(row3)
    deg = dparts[0, :, 0] + dparts[1, :, 0] + 1.0
    s = lax.rsqrt(deg)
    S = jnp.broadcast_to(s[:, None], (NPAD, D))

    g0 = _tc_lin(xp, wt1, b1r, S)
    p = _sc_propagate(g0, row3, col4)
    h1, g1 = _tc_mid(p, g0, S)
    p = _sc_propagate(g1, row3, col4)
    g0b = _tc_relu_lin(hw1, p, g1, h1, S, wt2, b2r)
    p = _sc_propagate(g0b, row3, col4)
    h1b, g1b = _tc_mid(p, g0b, S)
    p = _sc_propagate(g1b, row3, col4)
    out = _tc_final(hw2, p, g1b, h1b, S)
    return out[:N]


# speedup vs baseline: 1.3454x; 1.3454x over previous
import functools

import jax
import jax.numpy as jnp
from jax import lax
from jax.experimental import pallas as pl
from jax.experimental.pallas import tpu as pltpu
from jax.experimental.pallas import tpu_sc as plsc

N = 10000
E = 320000
D = 128
NPAD = 10240
NC, NS = 2, 16
NW = NC * NS
EPW = E // NW
CHUNK = 128
NCH = -(-EPW // CHUNK)
EPWP = NCH * CHUNK
RPS = NPAD // NS

_MESH = plsc.VectorSubcoreMesh(core_axis_name="c", subcore_axis_name="s")


def _zero_f32_buf(buf, nrows, ncols):
    def zr(i, _):
        for j in range(ncols // 16):
            buf[i, pl.ds(j * 16, 16)] = jnp.zeros((16,), jnp.float32)
        return _
    lax.fori_loop(0, nrows, zr, None)


def _prop_body(g_hbm, row_hbm, col_hbm, out_hbm, acc, ri, ci_a, ci_b,
               rows_a, rows_b, gsem_a, gsem_b, csem_a, csem_b):
    cid = lax.axis_index("c")
    sid = lax.axis_index("s")
    w = cid * NS + sid

    _zero_f32_buf(rows_a, CHUNK, D)
    base = sid * RPS
    for k in range(RPS // CHUNK):
        pltpu.sync_copy(rows_a, acc.at[pl.ds(base + k * CHUNK, CHUNK)])

    pltpu.sync_copy(row_hbm.at[w], ri)

    plsc.subcore_barrier()

    def gstart(c, rows, sem):
        cc = jnp.minimum(c, NCH - 1)
        return pltpu.async_copy(g_hbm.at[ri.at[cc]], rows, sem)

    def gwait(c, rows, sem):
        cc = jnp.minimum(c, NCH - 1)
        pltpu.make_async_copy(g_hbm.at[ri.at[cc]], rows, sem).wait()

    def cstart(c, buf, sem):
        cc = jnp.minimum(c, NCH - 1)
        return pltpu.async_copy(col_hbm.at[w, cc], buf, sem)

    def cwait(c, buf, sem):
        cc = jnp.minimum(c, NCH - 1)
        pltpu.make_async_copy(col_hbm.at[w, cc], buf, sem).wait()

    def scat(rows, ci):
        pltpu.sync_copy(rows, acc.at[ci.at[0]], add=True)

    def body(c, carry):
        cstart(c, ci_a, csem_a)
        gstart(c, rows_a, gsem_a)
        gwait(c, rows_a, gsem_a)
        cwait(c, ci_a, csem_a)
        scat(rows_a, ci_a)
        return carry

    lax.fori_loop(0, NCH, body, None)

    plsc.subcore_barrier()

    pltpu.sync_copy(acc.at[pl.ds(base, RPS)], out_hbm.at[cid, pl.ds(base, RPS)])


@functools.partial(
    pl.kernel,
    out_type=jax.ShapeDtypeStruct((NC, NPAD, D), jnp.float32),
    mesh=_MESH,
    scratch_types=[
        pltpu.VMEM_SHARED((NPAD, D), jnp.float32),
        pltpu.VMEM((NCH, CHUNK), jnp.int32),
        pltpu.VMEM((1, CHUNK), jnp.int32),
        pltpu.VMEM((1, CHUNK), jnp.int32),
        pltpu.VMEM((CHUNK, D), jnp.float32),
        pltpu.VMEM((CHUNK, D), jnp.float32),
        pltpu.SemaphoreType.DMA,
        pltpu.SemaphoreType.DMA,
        pltpu.SemaphoreType.DMA,
        pltpu.SemaphoreType.DMA,
    ],
)
def _sc_propagate(g_hbm, row_hbm, col_hbm, out_hbm, acc, ri, ci_a, ci_b,
                  rows_a, rows_b, gsem_a, gsem_b, csem_a, csem_b):
    _prop_body(g_hbm, row_hbm, col_hbm, out_hbm, acc, ri, ci_a, ci_b,
               rows_a, rows_b, gsem_a, gsem_b, csem_a, csem_b)


def _deg_body(row_hbm, out_hbm, dacc, ri, ones_v, zbuf):
    cid = lax.axis_index("c")
    sid = lax.axis_index("s")
    w = cid * NS + sid

    _zero_f32_buf(zbuf, 128, 16)
    base = sid * RPS
    for k in range(RPS // 128):
        pltpu.sync_copy(zbuf, dacc.at[pl.ds(base + k * 128, 128)])

    def fill(i, _):
        ones_v[i, pl.ds(0, 16)] = jnp.ones((16,), jnp.float32)
        return _
    lax.fori_loop(0, CHUNK, fill, None)

    pltpu.sync_copy(row_hbm.at[w], ri)

    plsc.subcore_barrier()

    def body(c, carry):
        pltpu.sync_copy(ones_v, dacc.at[ri.at[c]], add=True)
        return carry

    lax.fori_loop(0, NCH, body, None)

    plsc.subcore_barrier()

    pltpu.sync_copy(dacc.at[pl.ds(base, RPS)], out_hbm.at[cid, pl.ds(base, RPS)])


@functools.partial(
    pl.kernel,
    out_type=jax.ShapeDtypeStruct((NC, NPAD, 16), jnp.float32),
    mesh=_MESH,
    scratch_types=[
        pltpu.VMEM_SHARED((NPAD, 16), jnp.float32),
        pltpu.VMEM((NCH, CHUNK), jnp.int32),
        pltpu.VMEM((CHUNK, 16), jnp.float32),
        pltpu.VMEM((128, 16), jnp.float32),
    ],
)
def _sc_degree(row_hbm, out_hbm, dacc, ri, ones_v, zbuf):
    _deg_body(row_hbm, out_hbm, dacc, ri, ones_v, zbuf)


BLK = 2048
GRID = NPAD // BLK


def _row_spec():
    return pl.BlockSpec((BLK, D), lambda i: (i, 0))


def _full_spec():
    return pl.BlockSpec((D, D), lambda i: (0, 0))


def _bias_spec():
    return pl.BlockSpec((1, D), lambda i: (0, 0))


def _pair_spec():
    return pl.BlockSpec((NC, BLK, D), lambda i: (0, i, 0))


def _lin_body(x_ref, wt_ref, b_ref, s_ref, o_ref):
    xl = jnp.dot(x_ref[...], wt_ref[...], preferred_element_type=jnp.float32)
    o_ref[...] = s_ref[...] * (xl + b_ref[...])


_tc_lin = pl.pallas_call(
    _lin_body,
    grid=(GRID,),
    in_specs=[_row_spec(), _full_spec(), _bias_spec(), _row_spec()],
    out_specs=_row_spec(),
    out_shape=jax.ShapeDtypeStruct((NPAD, D), jnp.float32),
)


def _mid_body(p_ref, g0_ref, s_ref, h_ref, g1_ref):
    t = p_ref[0] + p_ref[1] + g0_ref[...]
    h = s_ref[...] * t
    h_ref[...] = h
    g1_ref[...] = s_ref[...] * h


_tc_mid = pl.pallas_call(
    _mid_body,
    grid=(GRID,),
    in_specs=[_pair_spec(), _row_spec(), _row_spec()],
    out_specs=[_row_spec(), _row_spec()],
    out_shape=[
        jax.ShapeDtypeStruct((NPAD, D), jnp.float32),
        jax.ShapeDtypeStruct((NPAD, D), jnp.float32),
    ],
)


def _relu_lin_body(hw_ref, p_ref, g1_ref, h1_ref, s_ref, wt_ref, b_ref, o_ref):
    h2 = s_ref[...] * (p_ref[0] + p_ref[1] + g1_ref[...])
    o = hw_ref[0] * h1_ref[...] + hw_ref[1] * h2
    o = jnp.maximum(o, 0.0)
    xl = jnp.dot(o, wt_ref[...], preferred_element_type=jnp.float32)
    o_ref[...] = s_ref[...] * (xl + b_ref[...])


_tc_relu_lin = pl.pallas_call(
    _relu_lin_body,
    grid=(GRID,),
    in_specs=[
        pl.BlockSpec(memory_space=pltpu.SMEM),
        _pair_spec(),
        _row_spec(),
        _row_spec(),
        _row_spec(),
        _full_spec(),
        _bias_spec(),
    ],
    out_specs=_row_spec(),
    out_shape=jax.ShapeDtypeStruct((NPAD, D), jnp.float32),
)


def _final_body(hw_ref, p_ref, g1_ref, h1_ref, s_ref, o_ref):
    h2 = s_ref[...] * (p_ref[0] + p_ref[1] + g1_ref[...])
    o_ref[...] = hw_ref[0] * h1_ref[...] + hw_ref[1] * h2


_tc_final = pl.pallas_call(
    _final_body,
    grid=(GRID,),
    in_specs=[
        pl.BlockSpec(memory_space=pltpu.SMEM),
        _pair_spec(),
        _row_spec(),
        _row_spec(),
        _row_spec(),
    ],
    out_specs=_row_spec(),
    out_shape=jax.ShapeDtypeStruct((NPAD, D), jnp.float32),
)


def kernel(x, edge_index, W1, b1, hw1, W2, b2, hw2):
    pad = jnp.full((NW, EPWP - EPW), N, dtype=jnp.int32)
    row_p = jnp.concatenate([edge_index[0].reshape(NW, EPW), pad], axis=1)
    col_p = jnp.concatenate([edge_index[1].reshape(NW, EPW), pad], axis=1)
    row3 = row_p.reshape(NW, NCH, CHUNK)
    col4 = col_p.reshape(NW, NCH, 1, CHUNK)

    xp = jnp.pad(x, ((0, NPAD - N), (0, 0)))
    b1r = b1.reshape(1, D)
    b2r = b2.reshape(1, D)
    wt1 = W1.T
    wt2 = W2.T

    dparts = _sc_degree(row3)
    deg = dparts[0, :, 0] + dparts[1, :, 0] + 1.0
    s = lax.rsqrt(deg)
    S = jnp.broadcast_to(s[:, None], (NPAD, D))

    g0 = _tc_lin(xp, wt1, b1r, S)
    p = _sc_propagate(g0, row3, col4)
    h1, g1 = _tc_mid(p, g0, S)
    p = _sc_propagate(g1, row3, col4)
    g0b = _tc_relu_lin(hw1, p, g1, h1, S, wt2, b2r)
    p = _sc_propagate(g0b, row3, col4)
    h1b, g1b = _tc_mid(p, g0b, S)
    p = _sc_propagate(g1b, row3, col4)
    out = _tc_final(hw2, p, g1b, h1b, S)
    return out[:N]


# speedup vs baseline: 2.1902x; 1.6279x over previous
import functools

import jax
import jax.numpy as jnp
from jax import lax
from jax.experimental import pallas as pl
from jax.experimental.pallas import tpu as pltpu
from jax.experimental.pallas import tpu_sc as plsc

N = 10000
E = 320000
D = 128
NPAD = 10240
NC, NS = 2, 16
NW = NC * NS
EPW = E // NW
CHUNK = 128
NCH = -(-EPW // CHUNK)
EPWP = NCH * CHUNK
RPS = NPAD // NS

_MESH = plsc.VectorSubcoreMesh(core_axis_name="c", subcore_axis_name="s")


def _zero_f32_buf(buf, nrows, ncols):
    def zr(i, _):
        for j in range(ncols // 16):
            buf[i, pl.ds(j * 16, 16)] = jnp.zeros((16,), jnp.float32)
        return _
    lax.fori_loop(0, nrows, zr, None)


def _prop_body(g_hbm, row_hbm, col_hbm, out_hbm, acc, ri, ci_a, ci_b,
               rows_a, rows_b, gsem_a, gsem_b, csem_a, csem_b):
    cid = lax.axis_index("c")
    sid = lax.axis_index("s")
    w = cid * NS + sid

    _zero_f32_buf(rows_a, CHUNK, D)
    base = sid * RPS
    for k in range(RPS // CHUNK):
        pltpu.sync_copy(rows_a, acc.at[pl.ds(base + k * CHUNK, CHUNK)])

    pltpu.sync_copy(row_hbm.at[w], ri)

    plsc.subcore_barrier()

    def gstart(c, rows, sem):
        cc = jnp.minimum(c, NCH - 1)
        return pltpu.async_copy(g_hbm.at[ri.at[cc]], rows, sem)

    def gwait(c, rows, sem):
        cc = jnp.minimum(c, NCH - 1)
        pltpu.make_async_copy(g_hbm.at[ri.at[cc]], rows, sem).wait()

    def cstart(c, buf, sem):
        cc = jnp.minimum(c, NCH - 1)
        return pltpu.async_copy(col_hbm.at[w, cc], buf, sem)

    def cwait(c, buf, sem):
        cc = jnp.minimum(c, NCH - 1)
        pltpu.make_async_copy(col_hbm.at[w, cc], buf, sem).wait()

    def scat(rows, ci):
        pltpu.sync_copy(rows, acc.at[ci.at[0]], add=True)

    def body(c, carry):
        cstart(c, ci_a, csem_a)
        gstart(c, rows_a, gsem_a)
        gwait(c, rows_a, gsem_a)
        cwait(c, ci_a, csem_a)
        scat(rows_a, ci_a)
        return carry

    lax.fori_loop(0, NCH, body, None)

    plsc.subcore_barrier()

    pltpu.sync_copy(acc.at[pl.ds(base, RPS)], out_hbm.at[cid, pl.ds(base, RPS)])


@functools.partial(
    pl.kernel,
    out_type=jax.ShapeDtypeStruct((NC, NPAD, D), jnp.float32),
    mesh=_MESH,
    scratch_types=[
        pltpu.VMEM_SHARED((NPAD, D), jnp.float32),
        pltpu.VMEM((NCH, CHUNK), jnp.int32),
        pltpu.VMEM((1, CHUNK), jnp.int32),
        pltpu.VMEM((1, CHUNK), jnp.int32),
        pltpu.VMEM((CHUNK, D), jnp.float32),
        pltpu.VMEM((CHUNK, D), jnp.float32),
        pltpu.SemaphoreType.DMA,
        pltpu.SemaphoreType.DMA,
        pltpu.SemaphoreType.DMA,
        pltpu.SemaphoreType.DMA,
    ],
)
def _sc_propagate(g_hbm, row_hbm, col_hbm, out_hbm, acc, ri, ci_a, ci_b,
                  rows_a, rows_b, gsem_a, gsem_b, csem_a, csem_b):
    _prop_body(g_hbm, row_hbm, col_hbm, out_hbm, acc, ri, ci_a, ci_b,
               rows_a, rows_b, gsem_a, gsem_b, csem_a, csem_b)


def _deg_body(row_hbm, out_hbm, dacc, ri, ones_v, zbuf):
    cid = lax.axis_index("c")
    sid = lax.axis_index("s")
    w = cid * NS + sid

    _zero_f32_buf(zbuf, 128, 16)
    base = sid * RPS
    for k in range(RPS // 128):
        pltpu.sync_copy(zbuf, dacc.at[pl.ds(base + k * 128, 128)])

    def fill(i, _):
        ones_v[i, pl.ds(0, 16)] = jnp.ones((16,), jnp.float32)
        return _
    lax.fori_loop(0, CHUNK, fill, None)

    pltpu.sync_copy(row_hbm.at[w], ri)

    plsc.subcore_barrier()

    def body(c, carry):
        pltpu.sync_copy(ones_v, dacc.at[ri.at[c]], add=True)
        return carry

    lax.fori_loop(0, NCH, body, None)

    plsc.subcore_barrier()

    pltpu.sync_copy(dacc.at[pl.ds(base, RPS)], out_hbm.at[cid, pl.ds(base, RPS)])


@functools.partial(
    pl.kernel,
    out_type=jax.ShapeDtypeStruct((NC, NPAD, 16), jnp.float32),
    mesh=_MESH,
    scratch_types=[
        pltpu.VMEM_SHARED((NPAD, 16), jnp.float32),
        pltpu.VMEM((NCH, CHUNK), jnp.int32),
        pltpu.VMEM((CHUNK, 16), jnp.float32),
        pltpu.VMEM((128, 16), jnp.float32),
    ],
)
def _sc_degree(row_hbm, out_hbm, dacc, ri, ones_v, zbuf):
    _deg_body(row_hbm, out_hbm, dacc, ri, ones_v, zbuf)


BLK = 2048
GRID = NPAD // BLK


def _row_spec():
    return pl.BlockSpec((BLK, D), lambda i: (i, 0))


def _full_spec():
    return pl.BlockSpec((D, D), lambda i: (0, 0))


def _bias_spec():
    return pl.BlockSpec((1, D), lambda i: (0, 0))


def _pair_spec():
    return pl.BlockSpec((NC, BLK, D), lambda i: (0, i, 0))


def _lin_body(x_ref, wt_ref, b_ref, s_ref, o_ref):
    xl = jnp.dot(x_ref[...], wt_ref[...], preferred_element_type=jnp.float32)
    o_ref[...] = s_ref[...] * (xl + b_ref[...])


_tc_lin = pl.pallas_call(
    _lin_body,
    grid=(GRID,),
    in_specs=[_row_spec(), _full_spec(), _bias_spec(), _row_spec()],
    out_specs=_row_spec(),
    out_shape=jax.ShapeDtypeStruct((NPAD, D), jnp.float32),
)


def _mid_body(p_ref, g0_ref, s_ref, h_ref, g1_ref):
    t = p_ref[0] + p_ref[1] + g0_ref[...]
    h = s_ref[...] * t
    h_ref[...] = h
    g1_ref[...] = s_ref[...] * h


_tc_mid = pl.pallas_call(
    _mid_body,
    grid=(GRID,),
    in_specs=[_pair_spec(), _row_spec(), _row_spec()],
    out_specs=[_row_spec(), _row_spec()],
    out_shape=[
        jax.ShapeDtypeStruct((NPAD, D), jnp.float32),
        jax.ShapeDtypeStruct((NPAD, D), jnp.float32),
    ],
)


def _relu_lin_body(hw_ref, p_ref, g1_ref, h1_ref, s_ref, wt_ref, b_ref, o_ref):
    h2 = s_ref[...] * (p_ref[0] + p_ref[1] + g1_ref[...])
    o = hw_ref[0] * h1_ref[...] + hw_ref[1] * h2
    o = jnp.maximum(o, 0.0)
    xl = jnp.dot(o, wt_ref[...], preferred_element_type=jnp.float32)
    o_ref[...] = s_ref[...] * (xl + b_ref[...])


_tc_relu_lin = pl.pallas_call(
    _relu_lin_body,
    grid=(GRID,),
    in_specs=[
        pl.BlockSpec(memory_space=pltpu.SMEM),
        _pair_spec(),
        _row_spec(),
        _row_spec(),
        _row_spec(),
        _full_spec(),
        _bias_spec(),
    ],
    out_specs=_row_spec(),
    out_shape=jax.ShapeDtypeStruct((NPAD, D), jnp.float32),
)


def _final_body(hw_ref, p_ref, g1_ref, h1_ref, s_ref, o_ref):
    h2 = s_ref[...] * (p_ref[0] + p_ref[1] + g1_ref[...])
    o_ref[...] = hw_ref[0] * h1_ref[...] + hw_ref[1] * h2


_tc_final = pl.pallas_call(
    _final_body,
    grid=(GRID,),
    in_specs=[
        pl.BlockSpec(memory_space=pltpu.SMEM),
        _pair_spec(),
        _row_spec(),
        _row_spec(),
        _row_spec(),
    ],
    out_specs=_row_spec(),
    out_shape=jax.ShapeDtypeStruct((NPAD, D), jnp.float32),
)


def kernel(x, edge_index, W1, b1, hw1, W2, b2, hw2):
    pad = N + jnp.arange(NW * (EPWP - EPW), dtype=jnp.int32) % (NPAD - N)
    pad = pad.reshape(NW, EPWP - EPW)
    row_p = jnp.concatenate([edge_index[0].reshape(NW, EPW), pad], axis=1)
    col_p = jnp.concatenate([edge_index[1].reshape(NW, EPW), pad], axis=1)
    row3 = row_p.reshape(NW, NCH, CHUNK)
    col4 = col_p.reshape(NW, NCH, 1, CHUNK)

    xp = jnp.pad(x, ((0, NPAD - N), (0, 0)))
    b1r = b1.reshape(1, D)
    b2r = b2.reshape(1, D)
    wt1 = W1.T
    wt2 = W2.T

    dparts = _sc_degree(row3)
    deg = dparts[0, :, 0] + dparts[1, :, 0] + 1.0
    s = lax.rsqrt(deg)
    S = jnp.broadcast_to(s[:, None], (NPAD, D))

    g0 = _tc_lin(xp, wt1, b1r, S)
    p = _sc_propagate(g0, row3, col4)
    h1, g1 = _tc_mid(p, g0, S)
    p = _sc_propagate(g1, row3, col4)
    g0b = _tc_relu_lin(hw1, p, g1, h1, S, wt2, b2r)
    p = _sc_propagate(g0b, row3, col4)
    h1b, g1b = _tc_mid(p, g0b, S)
    p = _sc_propagate(g1b, row3, col4)
    out = _tc_final(hw2, p, g1b, h1b, S)
    return out[:N]


# speedup vs baseline: 2.4560x; 1.1214x over previous
import functools

import jax
import jax.numpy as jnp
from jax import lax
from jax.experimental import pallas as pl
from jax.experimental.pallas import tpu as pltpu
from jax.experimental.pallas import tpu_sc as plsc

N = 10000
E = 320000
D = 128
NPAD = 10240
NC, NS = 2, 16
NW = NC * NS
EPW = E // NW
CHUNK = 128
NCH = -(-EPW // CHUNK)
EPWP = NCH * CHUNK
RPS = NPAD // NS

_MESH = plsc.VectorSubcoreMesh(core_axis_name="c", subcore_axis_name="s")


def _zero_f32_buf(buf, nrows, ncols):
    def zr(i, _):
        for j in range(ncols // 16):
            buf[i, pl.ds(j * 16, 16)] = jnp.zeros((16,), jnp.float32)
        return _
    lax.fori_loop(0, nrows, zr, None)


def _prop_body(g_hbm, row_hbm, col_hbm, out_hbm, acc, ri, ci_a, ci_b,
               rows_a, rows_b, gsem_a, gsem_b, csem_a, csem_b):
    cid = lax.axis_index("c")
    sid = lax.axis_index("s")
    w = cid * NS + sid

    _zero_f32_buf(rows_a, CHUNK, D)
    base = sid * RPS
    for k in range(RPS // CHUNK):
        pltpu.sync_copy(rows_a, acc.at[pl.ds(base + k * CHUNK, CHUNK)])

    pltpu.sync_copy(row_hbm.at[w], ri)

    plsc.subcore_barrier()

    def gstart(c, rows, sem):
        cc = jnp.minimum(c, NCH - 1)
        return pltpu.async_copy(g_hbm.at[ri.at[cc]], rows, sem)

    def gwait(c, rows, sem):
        cc = jnp.minimum(c, NCH - 1)
        pltpu.make_async_copy(g_hbm.at[ri.at[cc]], rows, sem).wait()

    def cstart(c, buf, sem):
        cc = jnp.minimum(c, NCH - 1)
        return pltpu.async_copy(col_hbm.at[w, cc], buf, sem)

    def cwait(c, buf, sem):
        cc = jnp.minimum(c, NCH - 1)
        pltpu.make_async_copy(col_hbm.at[w, cc], buf, sem).wait()

    def scat(rows, ci):
        pltpu.sync_copy(rows, acc.at[ci.at[0]], add=True)

    cstart(0, ci_a, csem_a)
    cstart(1, ci_b, csem_b)
    gstart(0, rows_a, gsem_a)
    gstart(1, rows_b, gsem_b)

    def body(cc, carry):
        e = 2 * cc
        gwait(e, rows_a, gsem_a)
        gwait(e + 1, rows_b, gsem_b)
        cwait(e, ci_a, csem_a)
        scat(rows_a, ci_a)
        cwait(e + 1, ci_b, csem_b)
        scat(rows_b, ci_b)
        cstart(e + 2, ci_a, csem_a)
        cstart(e + 3, ci_b, csem_b)
        gstart(e + 2, rows_a, gsem_a)
        gstart(e + 3, rows_b, gsem_b)
        return carry

    lax.fori_loop(0, NCH // 2, body, None)

    gwait(NCH - 1, rows_a, gsem_a)
    cwait(NCH - 1, ci_a, csem_a)
    scat(rows_a, ci_a)
    gwait(NCH, rows_b, gsem_b)
    cwait(NCH, ci_b, csem_b)

    plsc.subcore_barrier()

    pltpu.sync_copy(acc.at[pl.ds(base, RPS)], out_hbm.at[cid, pl.ds(base, RPS)])


@functools.partial(
    pl.kernel,
    out_type=jax.ShapeDtypeStruct((NC, NPAD, D), jnp.float32),
    mesh=_MESH,
    scratch_types=[
        pltpu.VMEM_SHARED((NPAD, D), jnp.float32),
        pltpu.VMEM((NCH, CHUNK), jnp.int32),
        pltpu.VMEM((1, CHUNK), jnp.int32),
        pltpu.VMEM((1, CHUNK), jnp.int32),
        pltpu.VMEM((CHUNK, D), jnp.float32),
        pltpu.VMEM((CHUNK, D), jnp.float32),
        pltpu.SemaphoreType.DMA,
        pltpu.SemaphoreType.DMA,
        pltpu.SemaphoreType.DMA,
        pltpu.SemaphoreType.DMA,
    ],
)
def _sc_propagate(g_hbm, row_hbm, col_hbm, out_hbm, acc, ri, ci_a, ci_b,
                  rows_a, rows_b, gsem_a, gsem_b, csem_a, csem_b):
    _prop_body(g_hbm, row_hbm, col_hbm, out_hbm, acc, ri, ci_a, ci_b,
               rows_a, rows_b, gsem_a, gsem_b, csem_a, csem_b)


def _deg_body(row_hbm, out_hbm, dacc, ri, ones_v, zbuf):
    cid = lax.axis_index("c")
    sid = lax.axis_index("s")
    w = cid * NS + sid

    _zero_f32_buf(zbuf, 128, 16)
    base = sid * RPS
    for k in range(RPS // 128):
        pltpu.sync_copy(zbuf, dacc.at[pl.ds(base + k * 128, 128)])

    def fill(i, _):
        ones_v[i, pl.ds(0, 16)] = jnp.ones((16,), jnp.float32)
        return _
    lax.fori_loop(0, CHUNK, fill, None)

    pltpu.sync_copy(row_hbm.at[w], ri)

    plsc.subcore_barrier()

    def body(c, carry):
        pltpu.sync_copy(ones_v, dacc.at[ri.at[c]], add=True)
        return carry

    lax.fori_loop(0, NCH, body, None)

    plsc.subcore_barrier()

    pltpu.sync_copy(dacc.at[pl.ds(base, RPS)], out_hbm.at[cid, pl.ds(base, RPS)])


@functools.partial(
    pl.kernel,
    out_type=jax.ShapeDtypeStruct((NC, NPAD, 16), jnp.float32),
    mesh=_MESH,
    scratch_types=[
        pltpu.VMEM_SHARED((NPAD, 16), jnp.float32),
        pltpu.VMEM((NCH, CHUNK), jnp.int32),
        pltpu.VMEM((CHUNK, 16), jnp.float32),
        pltpu.VMEM((128, 16), jnp.float32),
    ],
)
def _sc_degree(row_hbm, out_hbm, dacc, ri, ones_v, zbuf):
    _deg_body(row_hbm, out_hbm, dacc, ri, ones_v, zbuf)


BLK = 2048
GRID = NPAD // BLK


def _row_spec():
    return pl.BlockSpec((BLK, D), lambda i: (i, 0))


def _full_spec():
    return pl.BlockSpec((D, D), lambda i: (0, 0))


def _bias_spec():
    return pl.BlockSpec((1, D), lambda i: (0, 0))


def _pair_spec():
    return pl.BlockSpec((NC, BLK, D), lambda i: (0, i, 0))


def _lin_body(x_ref, wt_ref, b_ref, s_ref, o_ref):
    xl = jnp.dot(x_ref[...], wt_ref[...], preferred_element_type=jnp.float32)
    o_ref[...] = s_ref[...] * (xl + b_ref[...])


_tc_lin = pl.pallas_call(
    _lin_body,
    grid=(GRID,),
    in_specs=[_row_spec(), _full_spec(), _bias_spec(), _row_spec()],
    out_specs=_row_spec(),
    out_shape=jax.ShapeDtypeStruct((NPAD, D), jnp.float32),
)


def _mid_body(p_ref, g0_ref, s_ref, h_ref, g1_ref):
    t = p_ref[0] + p_ref[1] + g0_ref[...]
    h = s_ref[...] * t
    h_ref[...] = h
    g1_ref[...] = s_ref[...] * h


_tc_mid = pl.pallas_call(
    _mid_body,
    grid=(GRID,),
    in_specs=[_pair_spec(), _row_spec(), _row_spec()],
    out_specs=[_row_spec(), _row_spec()],
    out_shape=[
        jax.ShapeDtypeStruct((NPAD, D), jnp.float32),
        jax.ShapeDtypeStruct((NPAD, D), jnp.float32),
    ],
)


def _relu_lin_body(hw_ref, p_ref, g1_ref, h1_ref, s_ref, wt_ref, b_ref, o_ref):
    h2 = s_ref[...] * (p_ref[0] + p_ref[1] + g1_ref[...])
    o = hw_ref[0] * h1_ref[...] + hw_ref[1] * h2
    o = jnp.maximum(o, 0.0)
    xl = jnp.dot(o, wt_ref[...], preferred_element_type=jnp.float32)
    o_ref[...] = s_ref[...] * (xl + b_ref[...])


_tc_relu_lin = pl.pallas_call(
    _relu_lin_body,
    grid=(GRID,),
    in_specs=[
        pl.BlockSpec(memory_space=pltpu.SMEM),
        _pair_spec(),
        _row_spec(),
        _row_spec(),
        _row_spec(),
        _full_spec(),
        _bias_spec(),
    ],
    out_specs=_row_spec(),
    out_shape=jax.ShapeDtypeStruct((NPAD, D), jnp.float32),
)


def _final_body(hw_ref, p_ref, g1_ref, h1_ref, s_ref, o_ref):
    h2 = s_ref[...] * (p_ref[0] + p_ref[1] + g1_ref[...])
    o_ref[...] = hw_ref[0] * h1_ref[...] + hw_ref[1] * h2


_tc_final = pl.pallas_call(
    _final_body,
    grid=(GRID,),
    in_specs=[
        pl.BlockSpec(memory_space=pltpu.SMEM),
        _pair_spec(),
        _row_spec(),
        _row_spec(),
        _row_spec(),
    ],
    out_specs=_row_spec(),
    out_shape=jax.ShapeDtypeStruct((NPAD, D), jnp.float32),
)


def kernel(x, edge_index, W1, b1, hw1, W2, b2, hw2):
    pad = N + jnp.arange(NW * (EPWP - EPW), dtype=jnp.int32) % (NPAD - N)
    pad = pad.reshape(NW, EPWP - EPW)
    row_p = jnp.concatenate([edge_index[0].reshape(NW, EPW), pad], axis=1)
    col_p = jnp.concatenate([edge_index[1].reshape(NW, EPW), pad], axis=1)
    row3 = row_p.reshape(NW, NCH, CHUNK)
    col4 = col_p.reshape(NW, NCH, 1, CHUNK)

    xp = jnp.pad(x, ((0, NPAD - N), (0, 0)))
    b1r = b1.reshape(1, D)
    b2r = b2.reshape(1, D)
    wt1 = W1.T
    wt2 = W2.T

    dparts = _sc_degree(row3)
    deg = dparts[0, :, 0] + dparts[1, :, 0] + 1.0
    s = lax.rsqrt(deg)
    S = jnp.broadcast_to(s[:, None], (NPAD, D))

    g0 = _tc_lin(xp, wt1, b1r, S)
    p = _sc_propagate(g0, row3, col4)
    h1, g1 = _tc_mid(p, g0, S)
    p = _sc_propagate(g1, row3, col4)
    g0b = _tc_relu_lin(hw1, p, g1, h1, S, wt2, b2r)
    p = _sc_propagate(g0b, row3, col4)
    h1b, g1b = _tc_mid(p, g0b, S)
    p = _sc_propagate(g1b, row3, col4)
    out = _tc_final(hw2, p, g1b, h1b, S)
    return out[:N]
